# Initial kernel scaffold; baseline (speedup 1.0000x reference)
import jax
import jax.numpy as jnp
from jax.experimental import pallas as pl


def kernel(t_frames, t_units, coords, Omega, t_start_obs, t_geos, t_injection, grid):
    raise NotImplementedError("write your pallas kernel here")


# speedup vs baseline: 1.4040x; 1.4040x over previous
import functools

import jax
import jax.numpy as jnp
from jax import lax
from jax.experimental import pallas as pl
from jax.experimental.pallas import tpu as pltpu
from jax.experimental.pallas import tpu_sc as plsc

SCALE = 10.0
RMIN = 2.0
RMAX = 20.0
Z_WIDTH = 4.0
GRID_RES = 128

PG = GRID_RES + 3
PG2 = PG * PG
PGALL = PG * PG * PG

NW = 32
L = 16

NT, NA, NB, NG = 8, 64, 64, 64
NPTS = NT * NA * NB * NG
PPW = NPTS // NW
C = 2048
NSUB = PPW // C
MROWS = C // 128
UPG = 128 // L

_CORNER_OFF = (0, 1, PG, PG + 1, PG2, PG2 + 1, PG2 + PG, PG2 + PG + 1)

_TWO_OVER_PI = 0.6366197723675814


def _floorf(x):
    t = lax.convert_element_type(lax.convert_element_type(x, jnp.int32), jnp.float32)
    return jnp.where(t > x, t - 1.0, t)


def _sincos(th):
    kf = th * _TWO_OVER_PI
    kf = kf + jnp.where(kf >= 0.0, 0.5, -0.5)
    ki = lax.convert_element_type(kf, jnp.int32)
    kr = lax.convert_element_type(ki, jnp.float32)
    r = th - kr * 1.5707964
    r = r + kr * 4.3711388e-8
    r2 = r * r
    sinp = r * (1.0 + r2 * (-0.16666667 + r2 * (0.008333333 + r2 * -1.9841270e-4)))
    cosp = 1.0 + r2 * (-0.5 + r2 * (0.041666668 + r2 * -0.0013888889))
    q = lax.bitwise_and(ki, 3)
    odd = lax.bitwise_and(ki, 1) == 1
    sb = jnp.where(odd, cosp, sinp)
    cb = jnp.where(odd, sinp, cosp)
    s = jnp.where(q >= 2, -sb, sb)
    c = jnp.where((q == 1) | (q == 2), -cb, cb)
    return s, c


def _sc_body(coords_hbm, om_hbm, tg_hbm, tb_hbm, grid_hbm, out_hbm,
             xb0, yb0, zb0, ob0, gb0, wxb0, wyb0, wzb0, mb0, idxb0, gathb0, outb0,
             xb1, yb1, zb1, ob1, gb1, wxb1, wyb1, wzb1, mb1, idxb1, gathb1, outb1,
             tvm, sem_in, sem_g):
    cid = lax.axis_index("c")
    sid = lax.axis_index("s")
    wid = sid * 2 + cid
    wbase = wid * PPW

    bufs = (
        (xb0, yb0, zb0, ob0, gb0, wxb0, wyb0, wzb0, mb0, idxb0, gathb0, outb0),
        (xb1, yb1, zb1, ob1, gb1, wxb1, wyb1, wzb1, mb1, idxb1, gathb1, outb1),
    )

    pltpu.sync_copy(tb_hbm.at[wid], tvm)

    def in_copies(s, b):
        xb, yb, zb, ob, gb = bufs[b][:5]
        off = wbase + s * C
        return (
            (coords_hbm.at[pl.ds(off, C)], xb),
            (coords_hbm.at[pl.ds(NPTS + off, C)], yb),
            (coords_hbm.at[pl.ds(2 * NPTS + off, C)], zb),
            (om_hbm.at[pl.ds(off, C)], ob),
            (tg_hbm.at[pl.ds(off, C)], gb),
        )

    def fire_loads(s, b):
        for src, dst in in_copies(s, b):
            pltpu.async_copy(src, dst, sem_in)

    def drain_loads(s, b):
        for src, dst in in_copies(s, b):
            pltpu.make_async_copy(src, dst, sem_in).wait()

    def compute(b):
        xb, yb, zb, ob, gb, wxb, wyb, wzb, mb, idxb = bufs[b][:10]
        tvec = tvm[...]

        def compute_row(mi, _):
            for u in range(UPG):
                o = mi * 128 + u * L
                x = xb[pl.ds(o, L)]
                y = yb[pl.ds(o, L)]
                z = zb[pl.ds(o, L)]
                om = ob[pl.ds(o, L)]
                tg = gb[pl.ds(o, L)]
                trot = tvec + tg
                s_, c_ = _sincos(-(om * trot))
                xw = x * c_ + y * s_
                yw = y * c_ - x * s_
                gx = jnp.minimum(jnp.maximum(xw * 6.35 + 63.5, -1.0), 128.0)
                gy = jnp.minimum(jnp.maximum(yw * 6.35 + 63.5, -1.0), 128.0)
                gz = jnp.minimum(jnp.maximum(z * 6.35 + 63.5, -1.0), 128.0)
                fxf = _floorf(gx)
                fyf = _floorf(gy)
                fzf = _floorf(gz)
                ix = lax.convert_element_type(fxf, jnp.int32)
                iy = lax.convert_element_type(fyf, jnp.int32)
                iz = lax.convert_element_type(fzf, jnp.int32)
                base = ix * PG2 + iy * PG + iz + (PG2 + PG + 1)
                r2 = x * x + y * y + z * z
                keep = ((r2 >= RMIN * RMIN) & (r2 <= RMAX * RMAX)
                        & (jnp.abs(z) <= Z_WIDTH * 0.5) & (trot >= 0.0))
                wxb[pl.ds(o, L)] = gx - fxf
                wyb[pl.ds(o, L)] = gy - fyf
                wzb[pl.ds(o, L)] = gz - fzf
                mb[pl.ds(o, L)] = jnp.where(keep, 1.0, 0.0)
                for k in range(8):
                    idxb[pl.ds(k * C + o, L)] = base + _CORNER_OFF[k]
            return _

        lax.fori_loop(0, MROWS, compute_row, None)

    def fire_gather(b):
        idxb, gathb = bufs[b][9], bufs[b][10]
        pltpu.async_copy(grid_hbm.at[idxb], gathb, sem_g)

    def drain_gather(b):
        idxb, gathb = bufs[b][9], bufs[b][10]
        pltpu.make_async_copy(grid_hbm.at[idxb], gathb, sem_g).wait()

    def combine_store(s, b):
        wxb, wyb, wzb, mb, _i, gathb, outb = bufs[b][5:]

        def combine_row(mi, _):
            for u in range(UPG):
                o = mi * 128 + u * L
                fx = wxb[pl.ds(o, L)]
                fy = wyb[pl.ds(o, L)]
                fz = wzb[pl.ds(o, L)]
                mf = mb[pl.ds(o, L)]
                g = [gathb[pl.ds(k * C + o, L)] for k in range(8)]
                v00 = g[0] + fz * (g[1] - g[0])
                v01 = g[2] + fz * (g[3] - g[2])
                v10 = g[4] + fz * (g[5] - g[4])
                v11 = g[6] + fz * (g[7] - g[6])
                v0 = v00 + fy * (v01 - v00)
                v1 = v10 + fy * (v11 - v10)
                v = v0 + fx * (v1 - v0)
                xs = v - 10.0
                e = jnp.exp(-jnp.abs(xs))
                sig = jnp.where(xs >= 0.0, 1.0 / (1.0 + e), e / (1.0 + e))
                outb[pl.ds(o, L)] = mf * sig
            return _

        lax.fori_loop(0, MROWS, combine_row, None)
        pltpu.sync_copy(outb, out_hbm.at[pl.ds(wbase + s * C, C)])

    fire_loads(0, 0)

    def pair(i, _):
        s0 = 2 * i
        s1 = s0 + 1
        drain_loads(s0, 0)
        fire_loads(s1, 1)
        compute(0)
        fire_gather(0)

        @pl.when(i > 0)
        def _c1():
            drain_gather(1)
            combine_store(s0 - 1, 1)

        drain_loads(s1, 1)

        @pl.when(i < NSUB // 2 - 1)
        def _fl():
            fire_loads(s0 + 2, 0)

        compute(1)
        fire_gather(1)
        drain_gather(0)
        combine_store(s0, 0)
        return _

    lax.fori_loop(0, NSUB // 2, pair, None)
    drain_gather(1)
    combine_store(NSUB - 1, 1)


@jax.jit
def _run(coords2, om, tg, tb, gridp):
    mesh = plsc.VectorSubcoreMesh(core_axis_name="c", subcore_axis_name="s")
    per_parity = [
        pltpu.VMEM((C,), jnp.float32),
        pltpu.VMEM((C,), jnp.float32),
        pltpu.VMEM((C,), jnp.float32),
        pltpu.VMEM((C,), jnp.float32),
        pltpu.VMEM((C,), jnp.float32),
        pltpu.VMEM((C,), jnp.float32),
        pltpu.VMEM((C,), jnp.float32),
        pltpu.VMEM((C,), jnp.float32),
        pltpu.VMEM((C,), jnp.float32),
        pltpu.VMEM((8 * C,), jnp.int32),
        pltpu.VMEM((8 * C,), jnp.float32),
        pltpu.VMEM((C,), jnp.float32),
    ]
    f = functools.partial(
        pl.kernel,
        mesh=mesh,
        out_type=jax.ShapeDtypeStruct((NPTS,), jnp.float32),
        scratch_types=(per_parity + per_parity + [
            pltpu.VMEM((L,), jnp.float32),
            pltpu.SemaphoreType.DMA,
            pltpu.SemaphoreType.DMA,
        ]),
    )(_sc_body)
    return f(coords2, om, tg, tb, gridp)


def kernel(t_frames, t_units, coords, Omega, t_start_obs, t_geos, t_injection, grid):
    coords2 = coords.reshape(3 * NPTS)
    om = Omega.reshape(NPTS)
    tg = t_geos.reshape(NPTS)
    tadj = t_frames - t_start_obs[0] - t_injection[0]
    tb = jnp.broadcast_to(jnp.repeat(tadj, NW // NT)[:, None], (NW, L))
    gridp = jnp.pad(grid, ((1, 2), (1, 2), (1, 2))).reshape(PGALL)
    out = _run(coords2, om, tg, tb, gridp)
    return out.reshape(NT, NA, NB, NG)
